# baseline (speedup 1.0000x reference)
import jax
import jax.numpy as jnp
from jax.experimental import pallas as pl
from jax.experimental.pallas import tpu as pltpu

_N = 2048
_NFEAT = 128
_NHID = 8
_NCLASS = 32
_NHEADS = 8
_ALPHA = 0.2
_TILE_R = 128
_NTILES = _N // _TILE_R


def _elu(x):
    return jnp.where(x > 0, x, jnp.exp(x) - 1.0)


def _prep1_kernel(T_ref, Wh_ref, ah_ref, h_ref, cu1_ref, cu2_ref,
                  cv1_ref, cv2_ref):
    Tm = T_ref[...]
    ones = jnp.ones((_N, 1), dtype=jnp.bfloat16)
    for hd in range(_NHEADS):
        h = jnp.dot(Tm, Wh_ref[hd], preferred_element_type=jnp.float32,
                    precision=jax.lax.Precision.HIGHEST)
        h_ref[hd] = jnp.concatenate([h.astype(jnp.bfloat16), ones], axis=1)
    a1 = ah_ref[:, 0, :_NHID]
    a2 = ah_ref[:, 0, _NHID:]
    W1 = jnp.sum(Wh_ref[...] * a1[:, None, :], axis=2).T
    W2 = jnp.sum(Wh_ref[...] * a2[:, None, :], axis=2).T
    U = jnp.dot(Tm, W1, preferred_element_type=jnp.float32,
                precision=jax.lax.Precision.HIGHEST)
    V = jnp.dot(Tm, W2, preferred_element_type=jnp.float32,
                precision=jax.lax.Precision.HIGHEST)
    cu1_ref[...] = -U
    cu2_ref[...] = -_ALPHA * U
    Vt = V.T
    cv1_ref[...] = -Vt
    cv2_ref[...] = -_ALPHA * Vt


def _att1_kernel(adj_ref, h_ref, cu1_ref, cu2_ref, cv1_ref, cv2_ref, out_ref):
    adj_t = adj_ref[...]
    for hd in range(_NHEADS):
        nu1 = cu1_ref[:, hd:hd + 1]
        nu2 = cu2_ref[:, hd:hd + 1]
        nv1 = cv1_ref[hd:hd + 1, :]
        nv2 = cv2_ref[hd:hd + 1, :]
        arg = jnp.minimum(nu1 + nv1, nu2 + nv2)
        e = (jnp.exp(arg) * adj_t).astype(jnp.bfloat16)
        res = jnp.dot(e, h_ref[hd], preferred_element_type=jnp.float32)
        hp = res[:, :_NHID]
        rowsum = res[:, _NHID:_NHID + 1]
        out_ref[:, hd * _NHID:(hd + 1) * _NHID] = _elu(hp / rowsum)


def _prep2_kernel(x_ref, Wo_ref, ao_ref, h_ref, cu1_ref, cu2_ref,
                  cv1_ref, cv2_ref):
    xm = x_ref[...]
    h = jnp.dot(xm, Wo_ref[...], preferred_element_type=jnp.float32,
                precision=jax.lax.Precision.HIGHEST)
    ones = jnp.ones((_N, 1), dtype=jnp.bfloat16)
    h_ref[...] = jnp.concatenate([h.astype(jnp.bfloat16), ones], axis=1)
    a1 = ao_ref[0:1, :_NCLASS]
    a2 = ao_ref[0:1, _NCLASS:]
    W1 = jnp.sum(Wo_ref[...] * a1, axis=1, keepdims=True)
    W2 = jnp.sum(Wo_ref[...] * a2, axis=1, keepdims=True)
    U = jnp.dot(xm, W1, preferred_element_type=jnp.float32,
                precision=jax.lax.Precision.HIGHEST)
    V = jnp.dot(xm, W2, preferred_element_type=jnp.float32,
                precision=jax.lax.Precision.HIGHEST)
    cu1_ref[...] = -U
    cu2_ref[...] = -_ALPHA * U
    Vt = V.T
    cv1_ref[...] = -Vt
    cv2_ref[...] = -_ALPHA * Vt


def _att2_kernel(adj_ref, h_ref, cu1_ref, cu2_ref, cv1_ref, cv2_ref, out_ref):
    adj_t = adj_ref[...]
    nu1 = cu1_ref[...]
    nu2 = cu2_ref[...]
    nv1 = cv1_ref[...]
    nv2 = cv2_ref[...]
    arg = jnp.minimum(nu1 + nv1, nu2 + nv2)
    e = (jnp.exp(arg) * adj_t).astype(jnp.bfloat16)
    res = jnp.dot(e, h_ref[...], preferred_element_type=jnp.float32)
    hp = res[:, :_NCLASS]
    rowsum = res[:, _NCLASS:_NCLASS + 1]
    y = _elu(hp / rowsum)
    m = jnp.max(y, axis=1, keepdims=True)
    z = y - m
    lse = jnp.log(jnp.sum(jnp.exp(z), axis=1, keepdims=True))
    out_ref[...] = z - lse


def kernel(T, adj, W_heads, a_heads, W_out, a_out):
    f32 = jnp.float32
    bf16 = jnp.bfloat16

    h1, cu1, cu2, cv1, cv2 = pl.pallas_call(
        _prep1_kernel,
        out_shape=[
            jax.ShapeDtypeStruct((_NHEADS, _N, _NHID + 1), bf16),
            jax.ShapeDtypeStruct((_N, _NHEADS), f32),
            jax.ShapeDtypeStruct((_N, _NHEADS), f32),
            jax.ShapeDtypeStruct((_NHEADS, _N), f32),
            jax.ShapeDtypeStruct((_NHEADS, _N), f32),
        ],
    )(T, W_heads, a_heads)

    x1 = pl.pallas_call(
        _att1_kernel,
        grid=(_NTILES,),
        in_specs=[
            pl.BlockSpec((_TILE_R, _N), lambda i: (i, 0)),
            pl.BlockSpec((_NHEADS, _N, _NHID + 1), lambda i: (0, 0, 0)),
            pl.BlockSpec((_TILE_R, _NHEADS), lambda i: (i, 0)),
            pl.BlockSpec((_TILE_R, _NHEADS), lambda i: (i, 0)),
            pl.BlockSpec((_NHEADS, _N), lambda i: (0, 0)),
            pl.BlockSpec((_NHEADS, _N), lambda i: (0, 0)),
        ],
        out_specs=pl.BlockSpec((_TILE_R, _NHEADS * _NHID), lambda i: (i, 0)),
        out_shape=jax.ShapeDtypeStruct((_N, _NHEADS * _NHID), f32),
        compiler_params=pltpu.CompilerParams(
            dimension_semantics=("parallel",)),
    )(adj, h1, cu1, cu2, cv1, cv2)

    h2, du1, du2, dv1, dv2 = pl.pallas_call(
        _prep2_kernel,
        out_shape=[
            jax.ShapeDtypeStruct((_N, _NCLASS + 1), bf16),
            jax.ShapeDtypeStruct((_N, 1), f32),
            jax.ShapeDtypeStruct((_N, 1), f32),
            jax.ShapeDtypeStruct((1, _N), f32),
            jax.ShapeDtypeStruct((1, _N), f32),
        ],
    )(x1, W_out, a_out)

    out = pl.pallas_call(
        _att2_kernel,
        grid=(_NTILES,),
        in_specs=[
            pl.BlockSpec((_TILE_R, _N), lambda i: (i, 0)),
            pl.BlockSpec((_N, _NCLASS + 1), lambda i: (0, 0)),
            pl.BlockSpec((_TILE_R, 1), lambda i: (i, 0)),
            pl.BlockSpec((_TILE_R, 1), lambda i: (i, 0)),
            pl.BlockSpec((1, _N), lambda i: (0, 0)),
            pl.BlockSpec((1, _N), lambda i: (0, 0)),
        ],
        out_specs=pl.BlockSpec((_TILE_R, _NCLASS), lambda i: (i, 0)),
        out_shape=jax.ShapeDtypeStruct((_N, _NCLASS), f32),
        compiler_params=pltpu.CompilerParams(
            dimension_semantics=("parallel",)),
    )(adj, h2, du1, du2, dv1, dv2)
    return out


# speedup vs baseline: 1.2469x; 1.2469x over previous
import jax
import jax.numpy as jnp
from jax.experimental import pallas as pl
from jax.experimental.pallas import tpu as pltpu

_N = 2048
_NFEAT = 128
_NHID = 8
_NCLASS = 32
_NHEADS = 8
_ALPHA = 0.2
_TILE_R = 256
_NTILES = _N // _TILE_R


def _elu(x):
    return jnp.where(x > 0, x, jnp.exp(x) - 1.0)


def _layer1_kernel(adj_ref, T_ref, Wh_ref, ah_ref, out_ref,
                   h_s, cu1_s, cu2_s, cv1_s, cv2_s):
    i = pl.program_id(0)

    @pl.when(i == 0)
    def _prep():
        Tm = T_ref[...]
        ones = jnp.ones((_N, 1), dtype=jnp.bfloat16)
        for hd in range(_NHEADS):
            h = jnp.dot(Tm, Wh_ref[hd], preferred_element_type=jnp.float32,
                        precision=jax.lax.Precision.HIGHEST)
            h_s[hd] = jnp.concatenate([h.astype(jnp.bfloat16), ones], axis=1)
        a1 = ah_ref[:, 0, :_NHID]
        a2 = ah_ref[:, 0, _NHID:]
        W1 = jnp.sum(Wh_ref[...] * a1[:, None, :], axis=2).T
        W2 = jnp.sum(Wh_ref[...] * a2[:, None, :], axis=2).T
        U = jnp.dot(Tm, W1, preferred_element_type=jnp.float32,
                    precision=jax.lax.Precision.HIGHEST)
        V = jnp.dot(Tm, W2, preferred_element_type=jnp.float32,
                    precision=jax.lax.Precision.HIGHEST)
        cu1_s[...] = (-U).astype(jnp.bfloat16)
        cu2_s[...] = (-_ALPHA * U).astype(jnp.bfloat16)
        Vt = V.T
        cv1_s[...] = (-Vt).astype(jnp.bfloat16)
        cv2_s[...] = (-_ALPHA * Vt).astype(jnp.bfloat16)

    adj_t = adj_ref[...]
    for hd in range(_NHEADS):
        nu1 = cu1_s[pl.ds(i * _TILE_R, _TILE_R), hd:hd + 1]
        nu2 = cu2_s[pl.ds(i * _TILE_R, _TILE_R), hd:hd + 1]
        nv1 = cv1_s[hd:hd + 1, :]
        nv2 = cv2_s[hd:hd + 1, :]
        arg = jnp.minimum(nu1 + nv1, nu2 + nv2)
        e = jnp.exp(arg) * adj_t
        res = jnp.dot(e, h_s[hd], preferred_element_type=jnp.float32)
        hp = res[:, :_NHID]
        rowsum = res[:, _NHID:_NHID + 1]
        out_ref[:, hd * _NHID:(hd + 1) * _NHID] = _elu(hp / rowsum)


def _layer2_kernel(adj_ref, x_ref, Wo_ref, ao_ref, out_ref,
                   h_s, cu1_s, cu2_s, cv1_s, cv2_s):
    i = pl.program_id(0)

    @pl.when(i == 0)
    def _prep():
        xm = x_ref[...]
        h = jnp.dot(xm, Wo_ref[...], preferred_element_type=jnp.float32,
                    precision=jax.lax.Precision.HIGHEST)
        ones = jnp.ones((_N, 1), dtype=jnp.bfloat16)
        h_s[...] = jnp.concatenate([h.astype(jnp.bfloat16), ones], axis=1)
        a1 = ao_ref[0:1, :_NCLASS]
        a2 = ao_ref[0:1, _NCLASS:]
        W1 = jnp.sum(Wo_ref[...] * a1, axis=1, keepdims=True)
        W2 = jnp.sum(Wo_ref[...] * a2, axis=1, keepdims=True)
        U = jnp.dot(xm, W1, preferred_element_type=jnp.float32,
                    precision=jax.lax.Precision.HIGHEST)
        V = jnp.dot(xm, W2, preferred_element_type=jnp.float32,
                    precision=jax.lax.Precision.HIGHEST)
        cu1_s[...] = (-U).astype(jnp.bfloat16)
        cu2_s[...] = (-_ALPHA * U).astype(jnp.bfloat16)
        Vt = V.T
        cv1_s[...] = (-Vt).astype(jnp.bfloat16)
        cv2_s[...] = (-_ALPHA * Vt).astype(jnp.bfloat16)

    adj_t = adj_ref[...]
    nu1 = cu1_s[pl.ds(i * _TILE_R, _TILE_R), :]
    nu2 = cu2_s[pl.ds(i * _TILE_R, _TILE_R), :]
    nv1 = cv1_s[...]
    nv2 = cv2_s[...]
    arg = jnp.minimum(nu1 + nv1, nu2 + nv2)
    e = jnp.exp(arg) * adj_t
    res = jnp.dot(e, h_s[...], preferred_element_type=jnp.float32)
    hp = res[:, :_NCLASS]
    rowsum = res[:, _NCLASS:_NCLASS + 1]
    y = _elu(hp / rowsum)
    m = jnp.max(y, axis=1, keepdims=True)
    z = y - m
    lse = jnp.log(jnp.sum(jnp.exp(z), axis=1, keepdims=True))
    out_ref[...] = z - lse


def kernel(T, adj, W_heads, a_heads, W_out, a_out):
    f32 = jnp.float32
    bf16 = jnp.bfloat16
    adj_bf = adj.astype(bf16)

    x1 = pl.pallas_call(
        _layer1_kernel,
        grid=(_NTILES,),
        in_specs=[
            pl.BlockSpec((_TILE_R, _N), lambda i: (i, 0)),
            pl.BlockSpec((_N, _NFEAT), lambda i: (0, 0)),
            pl.BlockSpec((_NHEADS, _NFEAT, _NHID), lambda i: (0, 0, 0)),
            pl.BlockSpec((_NHEADS, 1, 2 * _NHID), lambda i: (0, 0, 0)),
        ],
        out_specs=pl.BlockSpec((_TILE_R, _NHEADS * _NHID), lambda i: (i, 0)),
        out_shape=jax.ShapeDtypeStruct((_N, _NHEADS * _NHID), f32),
        scratch_shapes=[
            pltpu.VMEM((_NHEADS, _N, _NHID + 1), bf16),
            pltpu.VMEM((_N, _NHEADS), bf16),
            pltpu.VMEM((_N, _NHEADS), bf16),
            pltpu.VMEM((_NHEADS, _N), bf16),
            pltpu.VMEM((_NHEADS, _N), bf16),
        ],
        compiler_params=pltpu.CompilerParams(
            dimension_semantics=("arbitrary",)),
    )(adj_bf, T, W_heads, a_heads)

    out = pl.pallas_call(
        _layer2_kernel,
        grid=(_NTILES,),
        in_specs=[
            pl.BlockSpec((_TILE_R, _N), lambda i: (i, 0)),
            pl.BlockSpec((_N, _NHEADS * _NHID), lambda i: (0, 0)),
            pl.BlockSpec((_NHEADS * _NHID, _NCLASS), lambda i: (0, 0)),
            pl.BlockSpec((1, 2 * _NCLASS), lambda i: (0, 0)),
        ],
        out_specs=pl.BlockSpec((_TILE_R, _NCLASS), lambda i: (i, 0)),
        out_shape=jax.ShapeDtypeStruct((_N, _NCLASS), f32),
        scratch_shapes=[
            pltpu.VMEM((_N, _NCLASS + 1), bf16),
            pltpu.VMEM((_N, 1), bf16),
            pltpu.VMEM((_N, 1), bf16),
            pltpu.VMEM((1, _N), bf16),
            pltpu.VMEM((1, _N), bf16),
        ],
        compiler_params=pltpu.CompilerParams(
            dimension_semantics=("arbitrary",)),
    )(adj_bf, x1, W_out, a_out)
    return out
